# baseline (speedup 1.0000x reference)
import jax
import jax.numpy as jnp
from jax.experimental import pallas as pl
from jax.experimental.pallas import tpu as pltpu


def _count_body(x_ref, t_ref, o_ref):
    C = x_ref.shape[1]
    Hb = x_ref.shape[2]
    CH = 8

    accs = [None] * C
    for i in range(Hb // CH):
        sl = slice(i * CH, (i + 1) * CH)
        t = t_ref[0, 0, sl, :]
        best = x_ref[0, 0, sl, :]
        pred = jnp.zeros(best.shape, jnp.int32)
        for c in range(1, C):
            xc = x_ref[0, c, sl, :]
            pred = jnp.where(xc > best, c, pred)
            best = jnp.maximum(xc, best)
        mval = jnp.where(pred == t, (1 << 20) + (1 << 10) + 1, 1 << 20)
        for c in range(C):
            v = jnp.where(t == c, mval, jnp.where(pred == c, 1 << 10, 0))
            accs[c] = v if i == 0 else accs[c] + v

    rows = [jnp.sum(a, axis=0, keepdims=True) for a in accs]
    o_ref[0, 0] = jnp.concatenate(rows, axis=0)


def _final_body(cnt_ref, o_ref):
    v = cnt_ref[...]
    mask = (1 << 10) - 1
    inter = jnp.sum((v & mask).astype(jnp.float32), axis=(1, 3))
    psum = jnp.sum(((v >> 10) & mask).astype(jnp.float32), axis=(1, 3))
    tsum = jnp.sum((v >> 20).astype(jnp.float32), axis=(1, 3))
    score = 2.0 * inter / (psum + tsum + 1e-10)
    o_ref[...] = jnp.mean(score, axis=0, keepdims=True)


def kernel(output, target):
    N, C, H, W = output.shape
    tgt = target.astype(jnp.int32)
    Hb = 128
    num_h = H // Hb

    cnt = pl.pallas_call(
        _count_body,
        grid=(N, num_h),
        in_specs=[
            pl.BlockSpec((1, C, Hb, W), lambda n, h: (n, 0, h, 0)),
            pl.BlockSpec((1, 1, Hb, W), lambda n, h: (n, 0, h, 0)),
        ],
        out_specs=pl.BlockSpec((1, 1, C, W), lambda n, h: (n, h, 0, 0)),
        out_shape=jax.ShapeDtypeStruct((N, num_h, C, W), jnp.int32),
        compiler_params=pltpu.CompilerParams(
            dimension_semantics=("parallel", "arbitrary"),
        ),
    )(output, tgt)

    out = pl.pallas_call(
        _final_body,
        out_shape=jax.ShapeDtypeStruct((1, C), jnp.float32),
    )(cnt)
    return out[0]


# speedup vs baseline: 1.0963x; 1.0963x over previous
import jax
import jax.numpy as jnp
from jax.experimental import pallas as pl
from jax.experimental.pallas import tpu as pltpu


def _count_body(x_ref, t_ref, o_ref):
    C = x_ref.shape[1]
    Hb = x_ref.shape[2]
    CH = 8
    NW = (C + 7) // 8
    EMASK = 0x0F0F0F0F

    p4 = [[None] * NW for _ in range(3)]
    p8 = [[None] * (2 * NW) for _ in range(3)]

    def fold():
        for h in range(3):
            for w in range(NW):
                a = p4[h][w]
                lo = a & EMASK
                hi = (a >> 4) & EMASK
                p8[h][2 * w] = lo if p8[h][2 * w] is None else p8[h][2 * w] + lo
                p8[h][2 * w + 1] = hi if p8[h][2 * w + 1] is None else p8[h][2 * w + 1] + hi
                p4[h][w] = None

    for i in range(Hb // CH):
        sl = slice(i * CH, (i + 1) * CH)
        t = t_ref[0, 0, sl, :]
        best = x_ref[0, 0, sl, :]
        pred = jnp.zeros(best.shape, jnp.int32)
        for c in range(1, C):
            xc = x_ref[0, c, sl, :]
            pred = jnp.where(xc > best, c, pred)
            best = jnp.maximum(xc, best)

        ct = 1 << ((t & 7) << 2)
        cp = 1 << ((pred & 7) << 2)
        ci = jnp.where(pred == t, ct, 0)
        tw = t >> 3
        pw = pred >> 3
        for w in range(NW):
            mt = tw == w
            vi = jnp.where(mt, ci, 0)
            vt = jnp.where(mt, ct, 0)
            vp = jnp.where(pw == w, cp, 0)
            for h, v in ((0, vi), (1, vp), (2, vt)):
                p4[h][w] = v if p4[h][w] is None else p4[h][w] + v
        if i % 8 == 7:
            fold()

    red = [[jnp.sum(a, axis=0, keepdims=True) for a in p8[h]] for h in range(3)]
    rows = []
    for c in range(C):
        w = 2 * (c >> 3) + (c & 1)
        sh = 8 * ((c & 7) >> 1)
        icnt = (red[0][w] >> sh) & 255
        pcnt = (red[1][w] >> sh) & 255
        tcnt = (red[2][w] >> sh) & 255
        rows.append(icnt + (pcnt << 10) + (tcnt << 20))
    o_ref[0, 0] = jnp.concatenate(rows, axis=0)


def _final_body(cnt_ref, o_ref):
    v = cnt_ref[...]
    mask = (1 << 10) - 1
    inter = jnp.sum((v & mask).astype(jnp.float32), axis=(1, 3))
    psum = jnp.sum(((v >> 10) & mask).astype(jnp.float32), axis=(1, 3))
    tsum = jnp.sum((v >> 20).astype(jnp.float32), axis=(1, 3))
    score = 2.0 * inter / (psum + tsum + 1e-10)
    o_ref[...] = jnp.mean(score, axis=0, keepdims=True)


def kernel(output, target):
    N, C, H, W = output.shape
    tgt = target.astype(jnp.int32)
    Hb = 128
    num_h = H // Hb

    cnt = pl.pallas_call(
        _count_body,
        grid=(N, num_h),
        in_specs=[
            pl.BlockSpec((1, C, Hb, W), lambda n, h: (n, 0, h, 0)),
            pl.BlockSpec((1, 1, Hb, W), lambda n, h: (n, 0, h, 0)),
        ],
        out_specs=pl.BlockSpec((1, 1, C, W), lambda n, h: (n, h, 0, 0)),
        out_shape=jax.ShapeDtypeStruct((N, num_h, C, W), jnp.int32),
        compiler_params=pltpu.CompilerParams(
            dimension_semantics=("parallel", "arbitrary"),
        ),
    )(output, tgt)

    out = pl.pallas_call(
        _final_body,
        out_shape=jax.ShapeDtypeStruct((1, C), jnp.float32),
    )(cnt)
    return out[0]


# speedup vs baseline: 1.1285x; 1.0294x over previous
import jax
import jax.numpy as jnp
from jax.experimental import pallas as pl
from jax.experimental.pallas import tpu as pltpu


def _body(x_ref, t_ref, o_ref, blk_ref, score_ref):
    C = x_ref.shape[1]
    Hb = x_ref.shape[2]
    CH = 8
    NW = (C + 7) // 8
    EMASK = 0x0F0F0F0F
    n = pl.program_id(0)
    h = pl.program_id(1)
    num_n = pl.num_programs(0)
    num_h = pl.num_programs(1)

    p4 = [[None] * NW for _ in range(3)]
    p8 = [[None] * (2 * NW) for _ in range(3)]

    def fold():
        for i3 in range(3):
            for w in range(NW):
                a = p4[i3][w]
                lo = a & EMASK
                hi = (a >> 4) & EMASK
                p8[i3][2 * w] = lo if p8[i3][2 * w] is None else p8[i3][2 * w] + lo
                p8[i3][2 * w + 1] = hi if p8[i3][2 * w + 1] is None else p8[i3][2 * w + 1] + hi
                p4[i3][w] = None

    for i in range(Hb // CH):
        sl = slice(i * CH, (i + 1) * CH)
        t = t_ref[0, 0, sl, :]
        best = x_ref[0, 0, sl, :]
        pred = jnp.zeros(best.shape, jnp.int32)
        for c in range(1, C):
            xc = x_ref[0, c, sl, :]
            pred = jnp.where(xc > best, c, pred)
            best = jnp.maximum(xc, best)

        ct = 1 << ((t & 7) << 2)
        cp = 1 << ((pred & 7) << 2)
        ci = jnp.where(pred == t, ct, 0)
        tw = t >> 3
        pw = pred >> 3
        for w in range(NW):
            mt = tw == w
            vi = jnp.where(mt, ci, 0)
            vt = jnp.where(mt, ct, 0)
            vp = jnp.where(pw == w, cp, 0)
            for i3, v in ((0, vi), (1, vp), (2, vt)):
                p4[i3][w] = v if p4[i3][w] is None else p4[i3][w] + v
        if i % 8 == 7:
            fold()

    red = [[jnp.sum(a, axis=0, keepdims=True) for a in p8[i3]] for i3 in range(3)]
    rows = []
    for c in range(C):
        w = 2 * (c >> 3) + (c & 1)
        sh = 8 * ((c & 7) >> 1)
        icnt = (red[0][w] >> sh) & 255
        pcnt = (red[1][w] >> sh) & 255
        tcnt = (red[2][w] >> sh) & 255
        rows.append(icnt + (pcnt << 10) + (tcnt << 20))
    upd = jnp.concatenate(rows, axis=0)

    @pl.when(h == 0)
    def _():
        blk_ref[...] = upd

    @pl.when(h != 0)
    def _():
        blk_ref[...] += upd

    @pl.when(h == num_h - 1)
    def _():
        v = blk_ref[...]
        mask = (1 << 10) - 1
        inter = jnp.sum((v & mask).astype(jnp.float32), axis=1, keepdims=True)
        psum = jnp.sum(((v >> 10) & mask).astype(jnp.float32), axis=1, keepdims=True)
        tsum = jnp.sum((v >> 20).astype(jnp.float32), axis=1, keepdims=True)
        score = 2.0 * inter / (psum + tsum + 1e-10)
        acc = jnp.where(n == 0, score, score_ref[...] + score)
        score_ref[...] = acc

        @pl.when(n == num_n - 1)
        def _():
            o_ref[...] = acc * (1.0 / num_n)


def kernel(output, target):
    N, C, H, W = output.shape
    tgt = target.astype(jnp.int32)
    Hb = 128
    num_h = H // Hb

    out = pl.pallas_call(
        _body,
        grid=(N, num_h),
        in_specs=[
            pl.BlockSpec((1, C, Hb, W), lambda n, h: (n, 0, h, 0)),
            pl.BlockSpec((1, 1, Hb, W), lambda n, h: (n, 0, h, 0)),
        ],
        out_specs=pl.BlockSpec((C, 1), lambda n, h: (0, 0)),
        out_shape=jax.ShapeDtypeStruct((C, 1), jnp.float32),
        scratch_shapes=[
            pltpu.VMEM((C, W), jnp.int32),
            pltpu.VMEM((C, 1), jnp.float32),
        ],
        compiler_params=pltpu.CompilerParams(
            dimension_semantics=("arbitrary", "arbitrary"),
        ),
    )(output, tgt)
    return out[:, 0]
